# Initial kernel scaffold; baseline (speedup 1.0000x reference)
import functools

import jax
import jax.numpy as jnp
from jax import lax
from jax.experimental import pallas as pl
from jax.experimental.pallas import tpu as pltpu
from jax.experimental.pallas import tpu_sc as plsc

B = 16384
F = 100
D = 32
NCAT = 10000
TOTAL = B * F

NC, NS, L = 2, 16, 16
NW = NC * NS
SPAN = TOTAL // NW

IDX_W = 128
CHUNK_ROWS = 8
CHUNK = CHUNK_ROWS * IDX_W
NCHUNK = SPAN // CHUNK
X_ROWS = TOTAL // IDX_W


def _tokenizer_gather(x2d, table):
    mesh = plsc.VectorSubcoreMesh(core_axis_name="c", subcore_axis_name="s")

    @functools.partial(
        pl.kernel,
        out_type=jax.ShapeDtypeStruct((X_ROWS, IDX_W, D), jnp.float32),
        mesh=mesh,
        scratch_types=[
            pltpu.VMEM((CHUNK_ROWS, IDX_W), jnp.int32),
            pltpu.VMEM((CHUNK_ROWS, IDX_W), jnp.int32),
            pltpu.VMEM((CHUNK_ROWS, IDX_W, D), jnp.float32),
            pltpu.SemaphoreType.DMA,
        ],
    )
    def k(x_hbm, table_hbm, out_hbm, xv, idxv, rows_v, sem):
        wid = lax.axis_index("s") * NC + lax.axis_index("c")
        row0 = wid * (SPAN // IDX_W)
        iota = lax.iota(jnp.int32, L)

        def chunk_body(g, _):
            row_start = row0 + g * CHUNK_ROWS
            pltpu.sync_copy(x_hbm.at[pl.ds(row_start, CHUNK_ROWS)], xv)
            flat0 = row_start * IDX_W
            for r in range(CHUNK_ROWS):
                for kk in range(IDX_W // L):
                    pos = iota + (flat0 + r * IDX_W + kk * L)
                    off = lax.rem(pos, F) * NCAT
                    sl = pl.ds(kk * L, L)
                    idxv[r, sl] = xv[r, sl] + off
            copies = [
                pltpu.make_async_copy(table_hbm.at[idxv.at[r]],
                                      rows_v.at[r], sem)
                for r in range(CHUNK_ROWS)
            ]
            for c in copies:
                c.start()
            for c in copies:
                c.wait()
            pltpu.sync_copy(rows_v, out_hbm.at[pl.ds(row_start, CHUNK_ROWS)])
            return 0

        lax.fori_loop(0, NCHUNK, chunk_body, 0)

    return k(x2d, table)


@jax.jit
def kernel(x, table, bias):
    del bias
    x2d = x.reshape(X_ROWS, IDX_W)
    out = _tokenizer_gather(x2d, table)
    return out.reshape(B, F, D)


# speedup vs baseline: 3.2221x; 3.2221x over previous
import functools

import jax
import jax.numpy as jnp
from jax import lax
from jax.experimental import pallas as pl
from jax.experimental.pallas import tpu as pltpu
from jax.experimental.pallas import tpu_sc as plsc

B = 16384
F = 100
D = 32
NCAT = 10000
TOTAL = B * F

NC, NS, L = 2, 16, 16
NW = NC * NS
SPAN = TOTAL // NW

IDX_W = 128
CHUNK_ROWS = 8
CHUNK = CHUNK_ROWS * IDX_W
NCHUNK = SPAN // CHUNK
X_ROWS = TOTAL // IDX_W


def _tokenizer_gather(x2d, table):
    mesh = plsc.VectorSubcoreMesh(core_axis_name="c", subcore_axis_name="s")

    @functools.partial(
        pl.kernel,
        out_type=jax.ShapeDtypeStruct((X_ROWS, IDX_W, D), jnp.float32),
        mesh=mesh,
        scratch_types=[
            pltpu.VMEM((CHUNK_ROWS, IDX_W), jnp.int32),
            pltpu.VMEM((CHUNK_ROWS, IDX_W), jnp.int32),
            pltpu.VMEM((CHUNK_ROWS, IDX_W, D), jnp.float32),
            pltpu.SemaphoreType.DMA,
        ],
        compiler_params=pltpu.CompilerParams(use_tc_tiling_on_sc=False),
    )
    def k(x_hbm, table_hbm, out_hbm, xv, idxv, rows_v, sem):
        wid = lax.axis_index("s") * NC + lax.axis_index("c")
        row0 = wid * (SPAN // IDX_W)
        iota = lax.iota(jnp.int32, L)

        def chunk_body(g, _):
            row_start = row0 + g * CHUNK_ROWS
            pltpu.sync_copy(x_hbm.at[pl.ds(row_start, CHUNK_ROWS)], xv)
            flat0 = row_start * IDX_W
            for r in range(CHUNK_ROWS):
                for kk in range(IDX_W // L):
                    pos = iota + (flat0 + r * IDX_W + kk * L)
                    off = lax.rem(pos, F) * NCAT
                    sl = pl.ds(kk * L, L)
                    idxv[r, sl] = xv[r, sl] + off
            copies = [
                pltpu.make_async_copy(table_hbm.at[idxv.at[r]],
                                      rows_v.at[r], sem)
                for r in range(CHUNK_ROWS)
            ]
            for c in copies:
                c.start()
            for c in copies:
                c.wait()
            pltpu.sync_copy(rows_v, out_hbm.at[pl.ds(row_start, CHUNK_ROWS)])
            return 0

        lax.fori_loop(0, NCHUNK, chunk_body, 0)

    return k(x2d, table)


@jax.jit
def kernel(x, table, bias):
    del bias
    x2d = x.reshape(X_ROWS, IDX_W)
    out = _tokenizer_gather(x2d, table)
    return out.reshape(B, F, D)
